# Initial kernel scaffold; baseline (speedup 1.0000x reference)
import math

import jax
import jax.numpy as jnp
from jax import lax
from jax.experimental import pallas as pl
from jax.experimental.pallas import tpu as pltpu
from jax.experimental.pallas import tpu_sc as plsc

F32 = jnp.float32
HD = 128
BN_EPS = 1e-5
_INV = 1.0 / math.sqrt(1.0 + BN_EPS)

NC, NSUB, NW = 2, 16, 32
NPAD_BIG = 10240
NPAD_SMALL = 512
EPAD_HET = 10240


def _cdiv(a, b):
    return (a + b - 1) // b


def _gelu(x):
    return 0.5 * x * (1.0 + lax.erf(x * (1.0 / math.sqrt(2.0))))


def _lrelu(x):
    return jnp.where(x >= 0, x, 0.2 * x)


def _mm_bias(x, W, b, rb=512):
    N, K = x.shape
    Ko = W.shape[1]

    def body(x_ref, w_ref, b_ref, o_ref):
        o_ref[...] = jnp.dot(x_ref[...], w_ref[...],
                             preferred_element_type=F32) + b_ref[...]

    return pl.pallas_call(
        body,
        grid=(_cdiv(N, rb),),
        in_specs=[pl.BlockSpec((rb, K), lambda i: (i, 0)),
                  pl.BlockSpec((K, Ko), lambda i: (0, 0)),
                  pl.BlockSpec((1, Ko), lambda i: (0, 0))],
        out_specs=pl.BlockSpec((rb, Ko), lambda i: (i, 0)),
        out_shape=jax.ShapeDtypeStruct((N, Ko), F32),
    )(x, W, b.reshape(1, Ko))


def _prep(x, bn1w, bn1b, Wl, bl, Wr, br, rb=512):
    N = x.shape[0]

    def body(x_ref, bw, bb, wl, blr, wr, brr, xl_ref, xr_ref):
        xbn = x_ref[...] * bw[...] + bb[...]
        xl_ref[...] = jnp.dot(xbn, wl[...], preferred_element_type=F32) + blr[...]
        xr_ref[...] = jnp.dot(xbn, wr[...], preferred_element_type=F32) + brr[...]

    full = lambda i: (0, 0)
    return pl.pallas_call(
        body,
        grid=(_cdiv(N, rb),),
        in_specs=[pl.BlockSpec((rb, HD), lambda i: (i, 0)),
                  pl.BlockSpec((1, HD), full), pl.BlockSpec((1, HD), full),
                  pl.BlockSpec((HD, HD), full), pl.BlockSpec((1, HD), full),
                  pl.BlockSpec((HD, HD), full), pl.BlockSpec((1, HD), full)],
        out_specs=[pl.BlockSpec((rb, HD), lambda i: (i, 0))] * 2,
        out_shape=[jax.ShapeDtypeStruct((N, HD), F32)] * 2,
    )(x, (bn1w * _INV).reshape(1, HD), bn1b.reshape(1, HD),
      Wl, bl.reshape(1, HD), Wr, br.reshape(1, HD))


def _edge_compute(gl, gr, ea16, We16, att, eb=2560):
    E = gl.shape[0]
    assert E % eb == 0

    def body(gl_ref, gr_ref, ea_ref, we_ref, att_ref, val_ref, exs_ref):
        glv = gl_ref[...]
        v = glv + gr_ref[...] + jnp.dot(ea_ref[...], we_ref[...],
                                        preferred_element_type=F32)
        h = _lrelu(v)
        ex = jnp.exp(jnp.sum(h * att_ref[...], axis=1, keepdims=True))
        val_ref[...] = ex * glv
        exs_ref[...] = jnp.broadcast_to(ex, (eb, 16))

    full = lambda i: (0, 0)
    return pl.pallas_call(
        body,
        grid=(E // eb,),
        in_specs=[pl.BlockSpec((eb, HD), lambda i: (i, 0)),
                  pl.BlockSpec((eb, HD), lambda i: (i, 0)),
                  pl.BlockSpec((eb, 16), lambda i: (i, 0)),
                  pl.BlockSpec((16, HD), full),
                  pl.BlockSpec((1, HD), full)],
        out_specs=[pl.BlockSpec((eb, HD), lambda i: (i, 0)),
                   pl.BlockSpec((eb, 16), lambda i: (i, 0))],
        out_shape=[jax.ShapeDtypeStruct((E, HD), F32),
                   jax.ShapeDtypeStruct((E, 16), F32)],
    )(gl, gr, ea16, We16, att.reshape(1, HD))


def _finalize(x, xl, xr, ea0, ea1, c0, c1, o0, o1, s0, s1,
              We16, att, bias, ebnw, ebnb, eW2, eb2, rb=512):
    N = x.shape[0]

    def body(x_ref, xl_ref, xr_ref, ea0r, ea1r, c0r, c1r, o0r, o1r, s0r, s1r,
             wer, attr, biasr, bnwr, bnbr, ew2r, eb2r, o_ref):
        xlv = xl_ref[...]
        easum = ea0r[...] + ea1r[...]
        cnt = c0r[...][:, 0:1] + c1r[...][:, 0:1]
        la = easum / jnp.maximum(cnt, 1.0)
        vl = xlv + xr_ref[...] + jnp.dot(la, wer[...], preferred_element_type=F32)
        exL = jnp.exp(jnp.sum(_lrelu(vl) * attr[...], axis=1, keepdims=True))
        s = s0r[...][:, 0:1] + s1r[...][:, 0:1] + exL + 1e-16
        gat = (o0r[...] + o1r[...] + exL * xlv) / s + biasr[...]
        z = x_ref[...] + gat
        zb = z * bnwr[...] + bnbr[...]
        z1 = _gelu(jnp.dot(zb, ew2r[...], preferred_element_type=F32) + eb2r[...])
        o_ref[...] = z + z1

    full = lambda i: (0, 0)
    blk = lambda i: (i, 0)
    return pl.pallas_call(
        body,
        grid=(_cdiv(N, rb),),
        in_specs=[pl.BlockSpec((rb, HD), blk), pl.BlockSpec((rb, HD), blk),
                  pl.BlockSpec((rb, HD), blk),
                  pl.BlockSpec((rb, 16), blk), pl.BlockSpec((rb, 16), blk),
                  pl.BlockSpec((rb, 16), blk), pl.BlockSpec((rb, 16), blk),
                  pl.BlockSpec((rb, HD), blk), pl.BlockSpec((rb, HD), blk),
                  pl.BlockSpec((rb, 16), blk), pl.BlockSpec((rb, 16), blk),
                  pl.BlockSpec((16, HD), full), pl.BlockSpec((1, HD), full),
                  pl.BlockSpec((1, HD), full), pl.BlockSpec((1, HD), full),
                  pl.BlockSpec((1, HD), full), pl.BlockSpec((HD, HD), full),
                  pl.BlockSpec((1, HD), full)],
        out_specs=pl.BlockSpec((rb, HD), blk),
        out_shape=jax.ShapeDtypeStruct((N, HD), F32),
    )(x, xl, xr, ea0, ea1, c0, c1, o0, o1, s0, s1,
      We16, att.reshape(1, HD), bias.reshape(1, HD),
      (ebnw * _INV).reshape(1, HD), ebnb.reshape(1, HD), eW2, eb2.reshape(1, HD))


def _het_fin(x, a0, a1, epsb, W1, b1, W2, b2, ebnw, ebnb, eW, eb, rb=512):
    N = x.shape[0]

    def body(x_ref, a0r, a1r, epsr, w1r, b1r, w2r, b2r, bnwr, bnbr, ewr, ebr,
             o_ref):
        xv = x_ref[...]
        h = xv * epsr[...] + a0r[...] + a1r[...]
        g = jnp.maximum(jnp.dot(h, w1r[...], preferred_element_type=F32)
                        + b1r[...], 0.0)
        g = jnp.dot(g, w2r[...], preferred_element_type=F32) + b2r[...]
        z = xv + g
        zb = z * bnwr[...] + bnbr[...]
        z1 = _gelu(jnp.dot(zb, ewr[...], preferred_element_type=F32) + ebr[...])
        o_ref[...] = z + z1

    full = lambda i: (0, 0)
    blk = lambda i: (i, 0)
    return pl.pallas_call(
        body,
        grid=(_cdiv(N, rb),),
        in_specs=[pl.BlockSpec((rb, HD), blk), pl.BlockSpec((rb, HD), blk),
                  pl.BlockSpec((rb, HD), blk), pl.BlockSpec((1, HD), full),
                  pl.BlockSpec((HD, HD), full), pl.BlockSpec((1, HD), full),
                  pl.BlockSpec((HD, HD), full), pl.BlockSpec((1, HD), full),
                  pl.BlockSpec((1, HD), full), pl.BlockSpec((1, HD), full),
                  pl.BlockSpec((HD, HD), full), pl.BlockSpec((1, HD), full)],
        out_specs=pl.BlockSpec((rb, HD), blk),
        out_shape=jax.ShapeDtypeStruct((N, HD), F32),
    )(x, a0, a1, epsb, W1, b1.reshape(1, HD), W2, b2.reshape(1, HD),
      (ebnw * _INV).reshape(1, HD), ebnb.reshape(1, HD), eW, eb.reshape(1, HD))


def _mix(a, b, w, bb, rb=512):
    N = a.shape[0]

    def body(ar, br, wr, bbr, o_ref):
        o_ref[...] = (ar[...] + br[...]) * wr[...] + bbr[...]

    full = lambda i: (0, 0)
    blk = lambda i: (i, 0)
    return pl.pallas_call(
        body,
        grid=(_cdiv(N, rb),),
        in_specs=[pl.BlockSpec((rb, HD), blk), pl.BlockSpec((rb, HD), blk),
                  pl.BlockSpec((1, HD), full), pl.BlockSpec((1, HD), full)],
        out_specs=pl.BlockSpec((rb, HD), blk),
        out_shape=jax.ShapeDtypeStruct((N, HD), F32),
    )(a, b, (w * (0.5 * _INV)).reshape(1, HD), bb.reshape(1, HD))


def _sc_mesh():
    return plsc.VectorSubcoreMesh(core_axis_name="c", subcore_axis_name="s",
                                  num_cores=NC, num_subcores=NSUB)


def _fill_const(ref, nrows, value):
    v = jnp.full((16,), value, F32)
    for r in range(nrows):
        ref[r] = v


def _fill_const128(ref, nrows, value):
    v = jnp.full((16,), value, F32)
    for r in range(nrows):
        for cc in range(HD // 16):
            ref[r, pl.ds(cc * 16, 16)] = v


def _sc_gatv2_gather(xl, xr, src, dst, ea16):
    E = src.shape[0]
    PW = E // NW
    CO, KI = 400, 80
    NI = CO // KI
    NO = PW // CO
    NPAD = NPAD_BIG
    RPT = NPAD // NSUB

    def body(xl_hbm, xr_hbm, src_hbm, dst_hbm, ea_hbm,
             gl_hbm, gr_hbm, eas_hbm, cnt_hbm,
             srcv, dstv, eav, onesv, glv, grv, zv,
             eatab, ctab, sem1, sem2):
        c = lax.axis_index("c")
        sid = lax.axis_index("s")
        wid = sid * NC + c
        _fill_const(zv, 16, 0.0)
        _fill_const(onesv, KI, 1.0)
        r0 = sid * RPT

        def zb(i, carry):
            pltpu.sync_copy(zv, eatab.at[pl.ds(r0 + i * 16, 16)])
            pltpu.sync_copy(zv, ctab.at[pl.ds(r0 + i * 16, 16)])
            return carry

        lax.fori_loop(0, RPT // 16, zb, 0)
        plsc.subcore_barrier()

        def outer(o, carry):
            base = wid * PW + o * CO
            for j in range(NI):
                pltpu.sync_copy(src_hbm.at[pl.ds(base + j * KI, KI)], srcv.at[j])
                pltpu.sync_copy(dst_hbm.at[pl.ds(base + j * KI, KI)], dstv.at[j])
            pltpu.sync_copy(ea_hbm.at[pl.ds(base, CO)], eav)
            cps = []
            for j in range(NI):
                cps.append(pltpu.async_copy(
                    xl_hbm.at[srcv.at[j]], glv.at[pl.ds(j * KI, KI)], sem1))
                cps.append(pltpu.async_copy(
                    xr_hbm.at[dstv.at[j]], grv.at[pl.ds(j * KI, KI)], sem2))
            for j in range(NI):
                pltpu.sync_copy(eav.at[pl.ds(j * KI, KI)],
                                eatab.at[dstv.at[j]], add=True)
                pltpu.sync_copy(onesv, ctab.at[dstv.at[j]], add=True)
            for cp in cps:
                cp.wait()
            pltpu.sync_copy(glv, gl_hbm.at[pl.ds(base, CO)])
            pltpu.sync_copy(grv, gr_hbm.at[pl.ds(base, CO)])
            return carry

        lax.fori_loop(0, NO, outer, 0)
        plsc.subcore_barrier()
        pltpu.sync_copy(eatab.at[pl.ds(r0, RPT)], eas_hbm.at[c, pl.ds(r0, RPT)])
        pltpu.sync_copy(ctab.at[pl.ds(r0, RPT)], cnt_hbm.at[c, pl.ds(r0, RPT)])

    f = pl.kernel(
        body,
        out_type=(jax.ShapeDtypeStruct((E, HD), F32),
                  jax.ShapeDtypeStruct((E, HD), F32),
                  jax.ShapeDtypeStruct((NC, NPAD, 16), F32),
                  jax.ShapeDtypeStruct((NC, NPAD, 16), F32)),
        mesh=_sc_mesh(),
        scratch_types=[pltpu.VMEM((NI, KI), jnp.int32),
                       pltpu.VMEM((NI, KI), jnp.int32),
                       pltpu.VMEM((CO, 16), F32),
                       pltpu.VMEM((KI, 16), F32),
                       pltpu.VMEM((CO, HD), F32),
                       pltpu.VMEM((CO, HD), F32),
                       pltpu.VMEM((16, 16), F32),
                       pltpu.VMEM_SHARED((NPAD, 16), F32),
                       pltpu.VMEM_SHARED((NPAD, 16), F32),
                       pltpu.SemaphoreType.DMA,
                       pltpu.SemaphoreType.DMA])
    return f(xl, xr, src, dst, ea16)


def _sc_scatter2(val, exs, dst):
    E = dst.shape[0]
    PW = E // NW
    CO, KI = 400, 80
    NI = CO // KI
    NO = PW // CO
    NPAD = NPAD_BIG
    RPT = NPAD // NSUB

    def body(val_hbm, exs_hbm, dst_hbm, out_hbm, s_hbm,
             valv, exv, dstv, zv128, zv16, outtab, stab):
        c = lax.axis_index("c")
        sid = lax.axis_index("s")
        wid = sid * NC + c
        _fill_const128(zv128, 16, 0.0)
        _fill_const(zv16, 16, 0.0)
        r0 = sid * RPT

        def zb(i, carry):
            pltpu.sync_copy(zv128, outtab.at[pl.ds(r0 + i * 16, 16)])
            pltpu.sync_copy(zv16, stab.at[pl.ds(r0 + i * 16, 16)])
            return carry

        lax.fori_loop(0, RPT // 16, zb, 0)
        plsc.subcore_barrier()

        def outer(o, carry):
            base = wid * PW + o * CO
            for j in range(NI):
                pltpu.sync_copy(dst_hbm.at[pl.ds(base + j * KI, KI)], dstv.at[j])
            pltpu.sync_copy(val_hbm.at[pl.ds(base, CO)], valv)
            pltpu.sync_copy(exs_hbm.at[pl.ds(base, CO)], exv)
            for j in range(NI):
                pltpu.sync_copy(valv.at[pl.ds(j * KI, KI)],
                                outtab.at[dstv.at[j]], add=True)
                pltpu.sync_copy(exv.at[pl.ds(j * KI, KI)],
                                stab.at[dstv.at[j]], add=True)
            return carry

        lax.fori_loop(0, NO, outer, 0)
        plsc.subcore_barrier()
        pltpu.sync_copy(outtab.at[pl.ds(r0, RPT)], out_hbm.at[c, pl.ds(r0, RPT)])
        pltpu.sync_copy(stab.at[pl.ds(r0, RPT)], s_hbm.at[c, pl.ds(r0, RPT)])

    f = pl.kernel(
        body,
        out_type=(jax.ShapeDtypeStruct((NC, NPAD, HD), F32),
                  jax.ShapeDtypeStruct((NC, NPAD, 16), F32)),
        mesh=_sc_mesh(),
        scratch_types=[pltpu.VMEM((CO, HD), F32),
                       pltpu.VMEM((CO, 16), F32),
                       pltpu.VMEM((NI, KI), jnp.int32),
                       pltpu.VMEM((16, HD), F32),
                       pltpu.VMEM((16, 16), F32),
                       pltpu.VMEM_SHARED((NPAD, HD), F32),
                       pltpu.VMEM_SHARED((NPAD, 16), F32)])
    return f(val, exs, dst)


def _sc_het(xsrc, srcp, dstp, npad):
    EP = srcp.shape[0]
    PW = EP // NW
    KI = 80
    NI = PW // KI
    RPT = npad // NSUB

    def body(xs_hbm, src_hbm, dst_hbm, agg_hbm,
             srcv, dstv, rows, zv128, tab, sem):
        c = lax.axis_index("c")
        sid = lax.axis_index("s")
        wid = sid * NC + c
        _fill_const128(zv128, 16, 0.0)
        r0 = sid * RPT

        def zb(i, carry):
            pltpu.sync_copy(zv128, tab.at[pl.ds(r0 + i * 16, 16)])
            return carry

        lax.fori_loop(0, RPT // 16, zb, 0)
        plsc.subcore_barrier()

        base = wid * PW
        for j in range(NI):
            pltpu.sync_copy(src_hbm.at[pl.ds(base + j * KI, KI)], srcv.at[j])
            pltpu.sync_copy(dst_hbm.at[pl.ds(base + j * KI, KI)], dstv.at[j])
        cps = [pltpu.async_copy(xs_hbm.at[srcv.at[j]],
                                rows.at[pl.ds(j * KI, KI)], sem)
               for j in range(NI)]
        for cp in cps:
            cp.wait()
        for j in range(NI):
            pltpu.sync_copy(rows.at[pl.ds(j * KI, KI)],
                            tab.at[dstv.at[j]], add=True)
        plsc.subcore_barrier()
        pltpu.sync_copy(tab.at[pl.ds(r0, RPT)], agg_hbm.at[c, pl.ds(r0, RPT)])

    f = pl.kernel(
        body,
        out_type=jax.ShapeDtypeStruct((NC, npad, HD), F32),
        mesh=_sc_mesh(),
        scratch_types=[pltpu.VMEM((NI, KI), jnp.int32),
                       pltpu.VMEM((NI, KI), jnp.int32),
                       pltpu.VMEM((PW, HD), F32),
                       pltpu.VMEM((16, HD), F32),
                       pltpu.VMEM_SHARED((npad, HD), F32),
                       pltpu.SemaphoreType.DMA])
    return f(xsrc, srcp, dstp)


def _egret_full(x, src, dst, ea16, p, We16):
    N = x.shape[0]
    xl, xr = _prep(x, p['bn1_w'], p['bn1_b'], p['Wl'], p['bl'], p['Wr'], p['br'])
    gl, gr, eas, cnt = _sc_gatv2_gather(xl, xr, src, dst, ea16)
    val, exs = _edge_compute(gl, gr, ea16, We16, p['att'])
    out2, s2 = _sc_scatter2(val, exs, dst)
    return _finalize(x, xl, xr, eas[0, :N], eas[1, :N], cnt[0, :N], cnt[1, :N],
                     out2[0, :N], out2[1, :N], s2[0, :N], s2[1, :N],
                     We16, p['att'], p['bias'], p['ebn_w'], p['ebn_b'],
                     p['eW'], p['eb'])


def _het_block(x_src, x_dst, srcp, dstp, npad, p):
    Nd = x_dst.shape[0]
    agg = _sc_het(x_src, srcp, dstp, npad)
    epsb = jnp.broadcast_to(1.0 + p['eps'], (1, HD)).astype(F32)
    return _het_fin(x_dst, agg[0, :Nd], agg[1, :Nd], epsb,
                    p['W1'], p['b1'], p['W2'], p['b2'],
                    p['ebn_w'], p['ebn_b'], p['eW'], p['eb'])


def kernel(x_protein, x_drug, x_supernode, edge_attr_pp, edge_attr_dd, params,
           edge_index_pp, edge_index_dd, edge_index_ps, edge_index_ds,
           edge_index_sp, edge_index_sd):
    n_p = x_protein.shape[0]
    n_d = x_drug.shape[0]
    n_s = x_supernode.shape[0]
    pr = params['proj']
    xp = _mm_bias(x_protein, pr['Wp'], pr['bp'])
    xd = _mm_bias(x_drug, pr['Wd'], pr['bd'])
    xs = _mm_bias(x_supernode, pr['Ws'], pr['bs'])

    ea_pp16 = edge_attr_pp
    ea_dd16 = jnp.pad(edge_attr_dd, ((0, 0), (0, 16 - edge_attr_dd.shape[1])))

    def pad_het(ei, dummy):
        ec = ei.shape[1]
        srcp = jnp.pad(ei[0], (0, EPAD_HET - ec))
        dstp = jnp.pad(ei[1], (0, EPAD_HET - ec), constant_values=dummy)
        return srcp, dstp

    ps_src, ps_dst = pad_het(edge_index_ps, n_s)
    ds_src, ds_dst = pad_het(edge_index_ds, n_s)
    sp_src, sp_dst = pad_het(edge_index_sp, n_p)
    sd_src, sd_dst = pad_het(edge_index_sd, n_d)

    for lp in params['layers']:
        We_dd16 = jnp.pad(lp['dd']['We'],
                          ((0, 16 - lp['dd']['We'].shape[0]), (0, 0)))
        o_pp = _egret_full(xp, edge_index_pp[0], edge_index_pp[1], ea_pp16,
                           lp['pp'], lp['pp']['We'])
        o_dd = _egret_full(xd, edge_index_dd[0], edge_index_dd[1], ea_dd16,
                           lp['dd'], We_dd16)
        o_ps = _het_block(xp, xs, ps_src, ps_dst, NPAD_SMALL, lp['ps'])
        o_ds = _het_block(xd, xs, ds_src, ds_dst, NPAD_SMALL, lp['ds'])
        o_sp = _het_block(xs, xp, sp_src, sp_dst, NPAD_BIG, lp['sp'])
        o_sd = _het_block(xs, xd, sd_src, sd_dst, NPAD_BIG, lp['sd'])
        xp = _mix(o_pp, o_sp, lp['bn_p'][0], lp['bn_p'][1])
        xd = _mix(o_dd, o_sd, lp['bn_d'][0], lp['bn_d'][1])
        xs = _mix(o_ps, o_ds, lp['bn_s'][0], lp['bn_s'][1])
    return xp, xd, xs


# speedup vs baseline: 5.7236x; 5.7236x over previous
import math

import jax
import jax.numpy as jnp
from jax import lax
from jax.experimental import pallas as pl
from jax.experimental.pallas import tpu as pltpu
from jax.experimental.pallas import tpu_sc as plsc

F32 = jnp.float32
HD = 128
BN_EPS = 1e-5
_INV = 1.0 / math.sqrt(1.0 + BN_EPS)

NC, NSUB, NW = 2, 16, 32
NPAD_BIG = 10240
NPAD_SMALL = 512
EPAD_HET = 10240


def _cdiv(a, b):
    return (a + b - 1) // b


def _gelu(x):
    return 0.5 * x * (1.0 + lax.erf(x * (1.0 / math.sqrt(2.0))))


def _lrelu(x):
    return jnp.where(x >= 0, x, 0.2 * x)


def _mm_bias(x, W, b, rb=512):
    N, K = x.shape
    Ko = W.shape[1]

    def body(x_ref, w_ref, b_ref, o_ref):
        o_ref[...] = jnp.dot(x_ref[...], w_ref[...],
                             preferred_element_type=F32) + b_ref[...]

    return pl.pallas_call(
        body,
        grid=(_cdiv(N, rb),),
        in_specs=[pl.BlockSpec((rb, K), lambda i: (i, 0)),
                  pl.BlockSpec((K, Ko), lambda i: (0, 0)),
                  pl.BlockSpec((1, Ko), lambda i: (0, 0))],
        out_specs=pl.BlockSpec((rb, Ko), lambda i: (i, 0)),
        out_shape=jax.ShapeDtypeStruct((N, Ko), F32),
    )(x, W, b.reshape(1, Ko))


def _prep(x, bn1w, bn1b, Wl, bl, Wr, br, rb=512):
    N = x.shape[0]

    def body(x_ref, bw, bb, wl, blr, wr, brr, xl_ref, xr_ref):
        xbn = x_ref[...] * bw[...] + bb[...]
        xl_ref[...] = jnp.dot(xbn, wl[...], preferred_element_type=F32) + blr[...]
        xr_ref[...] = jnp.dot(xbn, wr[...], preferred_element_type=F32) + brr[...]

    full = lambda i: (0, 0)
    return pl.pallas_call(
        body,
        grid=(_cdiv(N, rb),),
        in_specs=[pl.BlockSpec((rb, HD), lambda i: (i, 0)),
                  pl.BlockSpec((1, HD), full), pl.BlockSpec((1, HD), full),
                  pl.BlockSpec((HD, HD), full), pl.BlockSpec((1, HD), full),
                  pl.BlockSpec((HD, HD), full), pl.BlockSpec((1, HD), full)],
        out_specs=[pl.BlockSpec((rb, HD), lambda i: (i, 0))] * 2,
        out_shape=[jax.ShapeDtypeStruct((N, HD), F32)] * 2,
    )(x, (bn1w * _INV).reshape(1, HD), bn1b.reshape(1, HD),
      Wl, bl.reshape(1, HD), Wr, br.reshape(1, HD))


def _edge_compute(gl, gr, ea16, We16, att, eb=2560):
    E = gl.shape[0]
    assert E % eb == 0

    def body(gl_ref, gr_ref, ea_ref, we_ref, att_ref, val_ref, sm_ref):
        glv = gl_ref[...]
        eav = ea_ref[...]
        v = glv + gr_ref[...] + jnp.dot(eav, we_ref[...],
                                        preferred_element_type=F32)
        h = _lrelu(v)
        ex = jnp.exp(jnp.sum(h * att_ref[...], axis=1, keepdims=True))
        val_ref[...] = ex * glv
        one = jnp.ones((eb, 1), F32)
        sm_ref[...] = jnp.concatenate(
            [ex, eav, one, jnp.zeros((eb, HD - 18), F32)], axis=1)

    full = lambda i: (0, 0)
    return pl.pallas_call(
        body,
        grid=(E // eb,),
        in_specs=[pl.BlockSpec((eb, HD), lambda i: (i, 0)),
                  pl.BlockSpec((eb, HD), lambda i: (i, 0)),
                  pl.BlockSpec((eb, 16), lambda i: (i, 0)),
                  pl.BlockSpec((16, HD), full),
                  pl.BlockSpec((1, HD), full)],
        out_specs=[pl.BlockSpec((eb, HD), lambda i: (i, 0)),
                   pl.BlockSpec((eb, HD), lambda i: (i, 0))],
        out_shape=[jax.ShapeDtypeStruct((E, HD), F32),
                   jax.ShapeDtypeStruct((E, HD), F32)],
    )(gl, gr, ea16, We16, att.reshape(1, HD))


def _finalize(x, xl, xr, sm0, sm1, o0, o1,
              We16, att, bias, ebnw, ebnb, eW2, eb2, rb=512):
    N = x.shape[0]

    def body(x_ref, xl_ref, xr_ref, sm0r, sm1r, o0r, o1r,
             wer, attr, biasr, bnwr, bnbr, ew2r, eb2r, o_ref):
        xlv = xl_ref[...]
        sm = sm0r[...] + sm1r[...]
        easum = sm[:, 1:17]
        cnt = sm[:, 17:18]
        la = easum / jnp.maximum(cnt, 1.0)
        vl = xlv + xr_ref[...] + jnp.dot(la, wer[...], preferred_element_type=F32)
        exL = jnp.exp(jnp.sum(_lrelu(vl) * attr[...], axis=1, keepdims=True))
        s = sm[:, 0:1] + exL + 1e-16
        gat = (o0r[...] + o1r[...] + exL * xlv) / s + biasr[...]
        z = x_ref[...] + gat
        zb = z * bnwr[...] + bnbr[...]
        z1 = _gelu(jnp.dot(zb, ew2r[...], preferred_element_type=F32) + eb2r[...])
        o_ref[...] = z + z1

    full = lambda i: (0, 0)
    blk = lambda i: (i, 0)
    return pl.pallas_call(
        body,
        grid=(_cdiv(N, rb),),
        in_specs=[pl.BlockSpec((rb, HD), blk), pl.BlockSpec((rb, HD), blk),
                  pl.BlockSpec((rb, HD), blk),
                  pl.BlockSpec((rb, HD), blk), pl.BlockSpec((rb, HD), blk),
                  pl.BlockSpec((rb, HD), blk), pl.BlockSpec((rb, HD), blk),
                  pl.BlockSpec((16, HD), full), pl.BlockSpec((1, HD), full),
                  pl.BlockSpec((1, HD), full), pl.BlockSpec((1, HD), full),
                  pl.BlockSpec((1, HD), full), pl.BlockSpec((HD, HD), full),
                  pl.BlockSpec((1, HD), full)],
        out_specs=pl.BlockSpec((rb, HD), blk),
        out_shape=jax.ShapeDtypeStruct((N, HD), F32),
    )(x, xl, xr, sm0, sm1, o0, o1,
      We16, att.reshape(1, HD), bias.reshape(1, HD),
      (ebnw * _INV).reshape(1, HD), ebnb.reshape(1, HD), eW2, eb2.reshape(1, HD))


def _het_fin(x, a0, a1, epsb, W1, b1, W2, b2, ebnw, ebnb, eW, eb, rb=512):
    N = x.shape[0]

    def body(x_ref, a0r, a1r, epsr, w1r, b1r, w2r, b2r, bnwr, bnbr, ewr, ebr,
             o_ref):
        xv = x_ref[...]
        h = xv * epsr[...] + a0r[...] + a1r[...]
        g = jnp.maximum(jnp.dot(h, w1r[...], preferred_element_type=F32)
                        + b1r[...], 0.0)
        g = jnp.dot(g, w2r[...], preferred_element_type=F32) + b2r[...]
        z = xv + g
        zb = z * bnwr[...] + bnbr[...]
        z1 = _gelu(jnp.dot(zb, ewr[...], preferred_element_type=F32) + ebr[...])
        o_ref[...] = z + z1

    full = lambda i: (0, 0)
    blk = lambda i: (i, 0)
    return pl.pallas_call(
        body,
        grid=(_cdiv(N, rb),),
        in_specs=[pl.BlockSpec((rb, HD), blk), pl.BlockSpec((rb, HD), blk),
                  pl.BlockSpec((rb, HD), blk), pl.BlockSpec((1, HD), full),
                  pl.BlockSpec((HD, HD), full), pl.BlockSpec((1, HD), full),
                  pl.BlockSpec((HD, HD), full), pl.BlockSpec((1, HD), full),
                  pl.BlockSpec((1, HD), full), pl.BlockSpec((1, HD), full),
                  pl.BlockSpec((HD, HD), full), pl.BlockSpec((1, HD), full)],
        out_specs=pl.BlockSpec((rb, HD), blk),
        out_shape=jax.ShapeDtypeStruct((N, HD), F32),
    )(x, a0, a1, epsb, W1, b1.reshape(1, HD), W2, b2.reshape(1, HD),
      (ebnw * _INV).reshape(1, HD), ebnb.reshape(1, HD), eW, eb.reshape(1, HD))


def _mix(a, b, w, bb, rb=512):
    N = a.shape[0]

    def body(ar, br, wr, bbr, o_ref):
        o_ref[...] = (ar[...] + br[...]) * wr[...] + bbr[...]

    full = lambda i: (0, 0)
    blk = lambda i: (i, 0)
    return pl.pallas_call(
        body,
        grid=(_cdiv(N, rb),),
        in_specs=[pl.BlockSpec((rb, HD), blk), pl.BlockSpec((rb, HD), blk),
                  pl.BlockSpec((1, HD), full), pl.BlockSpec((1, HD), full)],
        out_specs=pl.BlockSpec((rb, HD), blk),
        out_shape=jax.ShapeDtypeStruct((N, HD), F32),
    )(a, b, (w * (0.5 * _INV)).reshape(1, HD), bb.reshape(1, HD))


def _sc_mesh():
    return plsc.VectorSubcoreMesh(core_axis_name="c", subcore_axis_name="s",
                                  num_cores=NC, num_subcores=NSUB)


def _fill_const128(ref, nrows, value):
    v = jnp.full((16,), value, F32)
    for r in range(nrows):
        for cc in range(HD // 16):
            ref[r, pl.ds(cc * 16, 16)] = v


def _sc_gatv2_gather(xl, xr, src, dst):
    E = src.shape[0]
    PW = E // NW
    CO, KI = 400, 80
    NI = CO // KI
    NO = PW // CO

    def body(xl_hbm, xr_hbm, src_hbm, dst_hbm,
             gl_hbm, gr_hbm,
             srcv, dstv, glv, grv, sem1, sem2):
        c = lax.axis_index("c")
        sid = lax.axis_index("s")
        wid = sid * NC + c

        def outer(o, carry):
            base = wid * PW + o * CO
            for j in range(NI):
                pltpu.sync_copy(src_hbm.at[pl.ds(base + j * KI, KI)], srcv.at[j])
                pltpu.sync_copy(dst_hbm.at[pl.ds(base + j * KI, KI)], dstv.at[j])
            cps = []
            for j in range(NI):
                cps.append(pltpu.async_copy(
                    xl_hbm.at[srcv.at[j]], glv.at[pl.ds(j * KI, KI)], sem1))
                cps.append(pltpu.async_copy(
                    xr_hbm.at[dstv.at[j]], grv.at[pl.ds(j * KI, KI)], sem2))
            for cp in cps:
                cp.wait()
            pltpu.sync_copy(glv, gl_hbm.at[pl.ds(base, CO)])
            pltpu.sync_copy(grv, gr_hbm.at[pl.ds(base, CO)])
            return carry

        lax.fori_loop(0, NO, outer, 0)

    f = pl.kernel(
        body,
        out_type=(jax.ShapeDtypeStruct((E, HD), F32),
                  jax.ShapeDtypeStruct((E, HD), F32)),
        mesh=_sc_mesh(),
        scratch_types=[pltpu.VMEM((NI, KI), jnp.int32),
                       pltpu.VMEM((NI, KI), jnp.int32),
                       pltpu.VMEM((CO, HD), F32),
                       pltpu.VMEM((CO, HD), F32),
                       pltpu.SemaphoreType.DMA,
                       pltpu.SemaphoreType.DMA])
    return f(xl, xr, src, dst)


def _sc_scatter_rows(val, dst, ncols, npad):
    E = dst.shape[0]
    PW = E // NW
    CO, KI = 80, 80
    NI = CO // KI
    NO = PW // CO
    RPT = npad // NSUB

    def body(val_hbm, dst_hbm, out_hbm, valv, dstv, zv, tab):
        c = lax.axis_index("c")
        sid = lax.axis_index("s")
        wid = sid * NC + c
        for r in range(16):
            for cc in range(ncols // 16):
                zv[r, pl.ds(cc * 16, 16)] = jnp.zeros((16,), F32)
        r0 = sid * RPT

        def zb(i, carry):
            pltpu.sync_copy(zv, tab.at[pl.ds(r0 + i * 16, 16)])
            return carry

        lax.fori_loop(0, RPT // 16, zb, 0)
        plsc.subcore_barrier()

        def outer(o, carry):
            base = wid * PW + o * CO
            for j in range(NI):
                pltpu.sync_copy(dst_hbm.at[pl.ds(base + j * KI, KI)], dstv.at[j])
            pltpu.sync_copy(val_hbm.at[pl.ds(base, CO)], valv)
            for j in range(NI):
                pltpu.sync_copy(valv.at[pl.ds(j * KI, KI)],
                                tab.at[dstv.at[j]], add=True)
            return carry

        lax.fori_loop(0, NO, outer, 0)
        plsc.subcore_barrier()
        pltpu.sync_copy(tab.at[pl.ds(r0, RPT)], out_hbm.at[c, pl.ds(r0, RPT)])

    f = pl.kernel(
        body,
        out_type=jax.ShapeDtypeStruct((NC, npad, ncols), F32),
        mesh=_sc_mesh(),
        scratch_types=[pltpu.VMEM((CO, ncols), F32),
                       pltpu.VMEM((NI, KI), jnp.int32),
                       pltpu.VMEM((16, ncols), F32),
                       pltpu.VMEM_SHARED((npad, ncols), F32)])
    return f(val, dst)


def _sc_het(xsrc, srcp, dstp, npad):
    EP = srcp.shape[0]
    PW = EP // NW
    KI = 80
    NI = PW // KI
    RPT = npad // NSUB

    def body(xs_hbm, src_hbm, dst_hbm, agg_hbm,
             srcv, dstv, rows, zv128, tab, sem):
        c = lax.axis_index("c")
        sid = lax.axis_index("s")
        wid = sid * NC + c
        _fill_const128(zv128, 16, 0.0)
        r0 = sid * RPT

        def zb(i, carry):
            pltpu.sync_copy(zv128, tab.at[pl.ds(r0 + i * 16, 16)])
            return carry

        lax.fori_loop(0, RPT // 16, zb, 0)
        plsc.subcore_barrier()

        base = wid * PW
        for j in range(NI):
            pltpu.sync_copy(src_hbm.at[pl.ds(base + j * KI, KI)], srcv.at[j])
            pltpu.sync_copy(dst_hbm.at[pl.ds(base + j * KI, KI)], dstv.at[j])
        for j in range(NI):
            pltpu.async_copy(xs_hbm.at[srcv.at[j]],
                             rows.at[pl.ds(j * KI, KI)], sem).wait()
            pltpu.sync_copy(rows.at[pl.ds(j * KI, KI)],
                            tab.at[dstv.at[j]], add=True)
        plsc.subcore_barrier()
        pltpu.sync_copy(tab.at[pl.ds(r0, RPT)], agg_hbm.at[c, pl.ds(r0, RPT)])

    f = pl.kernel(
        body,
        out_type=jax.ShapeDtypeStruct((NC, npad, HD), F32),
        mesh=_sc_mesh(),
        scratch_types=[pltpu.VMEM((NI, KI), jnp.int32),
                       pltpu.VMEM((NI, KI), jnp.int32),
                       pltpu.VMEM((PW, HD), F32),
                       pltpu.VMEM((16, HD), F32),
                       pltpu.VMEM_SHARED((npad, HD), F32),
                       pltpu.SemaphoreType.DMA])
    return f(xsrc, srcp, dstp)


def _egret_full(x, src, dst, ea16, p, We16):
    N = x.shape[0]
    xl, xr = _prep(x, p['bn1_w'], p['bn1_b'], p['Wl'], p['bl'], p['Wr'], p['br'])
    gl, gr = _sc_gatv2_gather(xl, xr, src, dst)
    val, sm = _edge_compute(gl, gr, ea16, We16, p['att'])
    out2 = _sc_scatter_rows(val, dst, HD, NPAD_BIG)
    sm2 = _sc_scatter_rows(sm, dst, HD, NPAD_BIG)
    return _finalize(x, xl, xr, sm2[0, :N], sm2[1, :N],
                     out2[0, :N], out2[1, :N],
                     We16, p['att'], p['bias'], p['ebn_w'], p['ebn_b'],
                     p['eW'], p['eb'])


def _het_block(x_src, x_dst, srcp, dstp, npad, p):
    Nd = x_dst.shape[0]
    agg = _sc_het(x_src, srcp, dstp, npad)
    epsb = jnp.broadcast_to(1.0 + p['eps'], (1, HD)).astype(F32)
    return _het_fin(x_dst, agg[0, :Nd], agg[1, :Nd], epsb,
                    p['W1'], p['b1'], p['W2'], p['b2'],
                    p['ebn_w'], p['ebn_b'], p['eW'], p['eb'])


def kernel(x_protein, x_drug, x_supernode, edge_attr_pp, edge_attr_dd, params,
           edge_index_pp, edge_index_dd, edge_index_ps, edge_index_ds,
           edge_index_sp, edge_index_sd):
    n_p = x_protein.shape[0]
    n_d = x_drug.shape[0]
    n_s = x_supernode.shape[0]
    pr = params['proj']
    xp = _mm_bias(x_protein, pr['Wp'], pr['bp'])
    xd = _mm_bias(x_drug, pr['Wd'], pr['bd'])
    xs = _mm_bias(x_supernode, pr['Ws'], pr['bs'])

    ea_pp16 = edge_attr_pp
    ea_dd16 = jnp.pad(edge_attr_dd, ((0, 0), (0, 16 - edge_attr_dd.shape[1])))

    def pad_het(ei, dummy):
        ec = ei.shape[1]
        srcp = jnp.pad(ei[0], (0, EPAD_HET - ec))
        dstp = jnp.pad(ei[1], (0, EPAD_HET - ec), constant_values=dummy)
        return srcp, dstp

    ps_src, ps_dst = pad_het(edge_index_ps, n_s)
    ds_src, ds_dst = pad_het(edge_index_ds, n_s)
    sp_src, sp_dst = pad_het(edge_index_sp, n_p)
    sd_src, sd_dst = pad_het(edge_index_sd, n_d)

    for lp in params['layers']:
        We_dd16 = jnp.pad(lp['dd']['We'],
                          ((0, 16 - lp['dd']['We'].shape[0]), (0, 0)))
        o_pp = _egret_full(xp, edge_index_pp[0], edge_index_pp[1], ea_pp16,
                           lp['pp'], lp['pp']['We'])
        o_dd = _egret_full(xd, edge_index_dd[0], edge_index_dd[1], ea_dd16,
                           lp['dd'], We_dd16)
        o_ps = _het_block(xp, xs, ps_src, ps_dst, NPAD_SMALL, lp['ps'])
        o_ds = _het_block(xd, xs, ds_src, ds_dst, NPAD_SMALL, lp['ds'])
        o_sp = _het_block(xs, xp, sp_src, sp_dst, NPAD_BIG, lp['sp'])
        o_sd = _het_block(xs, xd, sd_src, sd_dst, NPAD_BIG, lp['sd'])
        xp = _mix(o_pp, o_sp, lp['bn_p'][0], lp['bn_p'][1])
        xd = _mix(o_dd, o_sd, lp['bn_d'][0], lp['bn_d'][1])
        xs = _mix(o_ps, o_ds, lp['bn_s'][0], lp['bn_s'][1])
    return xp, xd, xs


# speedup vs baseline: 8.1661x; 1.4267x over previous
import math

import jax
import jax.numpy as jnp
from jax import lax
from jax.experimental import pallas as pl
from jax.experimental.pallas import tpu as pltpu
from jax.experimental.pallas import tpu_sc as plsc

F32 = jnp.float32
HD = 128
BN_EPS = 1e-5
_INV = 1.0 / math.sqrt(1.0 + BN_EPS)

NC, NSUB, NW = 2, 16, 32
NPAD_BIG = 10240
NPAD_SMALL = 512
EPAD_HET = 10240


def _cdiv(a, b):
    return (a + b - 1) // b


def _gelu(x):
    return 0.5 * x * (1.0 + lax.erf(x * (1.0 / math.sqrt(2.0))))


def _lrelu(x):
    return jnp.where(x >= 0, x, 0.2 * x)


def _mm_bias(x, W, b, rb=512):
    N, K = x.shape
    Ko = W.shape[1]

    def body(x_ref, w_ref, b_ref, o_ref):
        o_ref[...] = jnp.dot(x_ref[...], w_ref[...],
                             preferred_element_type=F32) + b_ref[...]

    return pl.pallas_call(
        body,
        grid=(_cdiv(N, rb),),
        in_specs=[pl.BlockSpec((rb, K), lambda i: (i, 0)),
                  pl.BlockSpec((K, Ko), lambda i: (0, 0)),
                  pl.BlockSpec((1, Ko), lambda i: (0, 0))],
        out_specs=pl.BlockSpec((rb, Ko), lambda i: (i, 0)),
        out_shape=jax.ShapeDtypeStruct((N, Ko), F32),
    )(x, W, b.reshape(1, Ko))


def _prep(x, bn1w, bn1b, Wl, bl, Wr, br, rb=512):
    N = x.shape[0]

    def body(x_ref, bw, bb, wl, blr, wr, brr, xl_ref, xr_ref):
        xbn = x_ref[...] * bw[...] + bb[...]
        xl_ref[...] = jnp.dot(xbn, wl[...], preferred_element_type=F32) + blr[...]
        xr_ref[...] = jnp.dot(xbn, wr[...], preferred_element_type=F32) + brr[...]

    full = lambda i: (0, 0)
    return pl.pallas_call(
        body,
        grid=(_cdiv(N, rb),),
        in_specs=[pl.BlockSpec((rb, HD), lambda i: (i, 0)),
                  pl.BlockSpec((1, HD), full), pl.BlockSpec((1, HD), full),
                  pl.BlockSpec((HD, HD), full), pl.BlockSpec((1, HD), full),
                  pl.BlockSpec((HD, HD), full), pl.BlockSpec((1, HD), full)],
        out_specs=[pl.BlockSpec((rb, HD), lambda i: (i, 0))] * 2,
        out_shape=[jax.ShapeDtypeStruct((N, HD), F32)] * 2,
    )(x, (bn1w * _INV).reshape(1, HD), bn1b.reshape(1, HD),
      Wl, bl.reshape(1, HD), Wr, br.reshape(1, HD))


def _edge_compute(gl, gr, ea16, We16, att, eb=2560):
    E = gl.shape[0]
    assert E % eb == 0

    def body(gl_ref, gr_ref, ea_ref, we_ref, att_ref, val_ref, sm_ref):
        glv = gl_ref[...]
        eav = ea_ref[...]
        v = glv + gr_ref[...] + jnp.dot(eav, we_ref[...],
                                        preferred_element_type=F32)
        h = _lrelu(v)
        ex = jnp.exp(jnp.sum(h * att_ref[...], axis=1, keepdims=True))
        val_ref[...] = ex * glv
        one = jnp.ones((eb, 1), F32)
        sm_ref[...] = jnp.concatenate(
            [ex, eav, one, jnp.zeros((eb, HD - 18), F32)], axis=1)

    full = lambda i: (0, 0)
    return pl.pallas_call(
        body,
        grid=(E // eb,),
        in_specs=[pl.BlockSpec((eb, HD), lambda i: (i, 0)),
                  pl.BlockSpec((eb, HD), lambda i: (i, 0)),
                  pl.BlockSpec((eb, 16), lambda i: (i, 0)),
                  pl.BlockSpec((16, HD), full),
                  pl.BlockSpec((1, HD), full)],
        out_specs=[pl.BlockSpec((eb, HD), lambda i: (i, 0)),
                   pl.BlockSpec((eb, HD), lambda i: (i, 0))],
        out_shape=[jax.ShapeDtypeStruct((E, HD), F32),
                   jax.ShapeDtypeStruct((E, HD), F32)],
    )(gl, gr, ea16, We16, att.reshape(1, HD))


def _finalize(x, xl, xr, sm0, sm1, o0, o1,
              We16, att, bias, ebnw, ebnb, eW2, eb2, rb=512):
    N = x.shape[0]

    def body(x_ref, xl_ref, xr_ref, sm0r, sm1r, o0r, o1r,
             wer, attr, biasr, bnwr, bnbr, ew2r, eb2r, o_ref):
        xlv = xl_ref[...]
        sm = sm0r[...] + sm1r[...]
        easum = sm[:, 1:17]
        cnt = sm[:, 17:18]
        la = easum / jnp.maximum(cnt, 1.0)
        vl = xlv + xr_ref[...] + jnp.dot(la, wer[...], preferred_element_type=F32)
        exL = jnp.exp(jnp.sum(_lrelu(vl) * attr[...], axis=1, keepdims=True))
        s = sm[:, 0:1] + exL + 1e-16
        gat = (o0r[...] + o1r[...] + exL * xlv) / s + biasr[...]
        z = x_ref[...] + gat
        zb = z * bnwr[...] + bnbr[...]
        z1 = _gelu(jnp.dot(zb, ew2r[...], preferred_element_type=F32) + eb2r[...])
        o_ref[...] = z + z1

    full = lambda i: (0, 0)
    blk = lambda i: (i, 0)
    return pl.pallas_call(
        body,
        grid=(_cdiv(N, rb),),
        in_specs=[pl.BlockSpec((rb, HD), blk), pl.BlockSpec((rb, HD), blk),
                  pl.BlockSpec((rb, HD), blk),
                  pl.BlockSpec((rb, HD), blk), pl.BlockSpec((rb, HD), blk),
                  pl.BlockSpec((rb, HD), blk), pl.BlockSpec((rb, HD), blk),
                  pl.BlockSpec((16, HD), full), pl.BlockSpec((1, HD), full),
                  pl.BlockSpec((1, HD), full), pl.BlockSpec((1, HD), full),
                  pl.BlockSpec((1, HD), full), pl.BlockSpec((HD, HD), full),
                  pl.BlockSpec((1, HD), full)],
        out_specs=pl.BlockSpec((rb, HD), blk),
        out_shape=jax.ShapeDtypeStruct((N, HD), F32),
    )(x, xl, xr, sm0, sm1, o0, o1,
      We16, att.reshape(1, HD), bias.reshape(1, HD),
      (ebnw * _INV).reshape(1, HD), ebnb.reshape(1, HD), eW2, eb2.reshape(1, HD))


def _het_fin(x, a0, a1, epsb, W1, b1, W2, b2, ebnw, ebnb, eW, eb, rb=512):
    N = x.shape[0]

    def body(x_ref, a0r, a1r, epsr, w1r, b1r, w2r, b2r, bnwr, bnbr, ewr, ebr,
             o_ref):
        xv = x_ref[...]
        h = xv * epsr[...] + a0r[...] + a1r[...]
        g = jnp.maximum(jnp.dot(h, w1r[...], preferred_element_type=F32)
                        + b1r[...], 0.0)
        g = jnp.dot(g, w2r[...], preferred_element_type=F32) + b2r[...]
        z = xv + g
        zb = z * bnwr[...] + bnbr[...]
        z1 = _gelu(jnp.dot(zb, ewr[...], preferred_element_type=F32) + ebr[...])
        o_ref[...] = z + z1

    full = lambda i: (0, 0)
    blk = lambda i: (i, 0)
    return pl.pallas_call(
        body,
        grid=(_cdiv(N, rb),),
        in_specs=[pl.BlockSpec((rb, HD), blk), pl.BlockSpec((rb, HD), blk),
                  pl.BlockSpec((rb, HD), blk), pl.BlockSpec((1, HD), full),
                  pl.BlockSpec((HD, HD), full), pl.BlockSpec((1, HD), full),
                  pl.BlockSpec((HD, HD), full), pl.BlockSpec((1, HD), full),
                  pl.BlockSpec((1, HD), full), pl.BlockSpec((1, HD), full),
                  pl.BlockSpec((HD, HD), full), pl.BlockSpec((1, HD), full)],
        out_specs=pl.BlockSpec((rb, HD), blk),
        out_shape=jax.ShapeDtypeStruct((N, HD), F32),
    )(x, a0, a1, epsb, W1, b1.reshape(1, HD), W2, b2.reshape(1, HD),
      (ebnw * _INV).reshape(1, HD), ebnb.reshape(1, HD), eW, eb.reshape(1, HD))


def _mix(a, b, w, bb, rb=512):
    N = a.shape[0]

    def body(ar, br, wr, bbr, o_ref):
        o_ref[...] = (ar[...] + br[...]) * wr[...] + bbr[...]

    full = lambda i: (0, 0)
    blk = lambda i: (i, 0)
    return pl.pallas_call(
        body,
        grid=(_cdiv(N, rb),),
        in_specs=[pl.BlockSpec((rb, HD), blk), pl.BlockSpec((rb, HD), blk),
                  pl.BlockSpec((1, HD), full), pl.BlockSpec((1, HD), full)],
        out_specs=pl.BlockSpec((rb, HD), blk),
        out_shape=jax.ShapeDtypeStruct((N, HD), F32),
    )(a, b, (w * (0.5 * _INV)).reshape(1, HD), bb.reshape(1, HD))


def _sc_mesh():
    return plsc.VectorSubcoreMesh(core_axis_name="c", subcore_axis_name="s",
                                  num_cores=NC, num_subcores=NSUB)


def _fill_const128(ref, nrows, value):
    v = jnp.full((16,), value, F32)
    for r in range(nrows):
        for cc in range(HD // 16):
            ref[r, pl.ds(cc * 16, 16)] = v


def _sc_gatv2_gather(xl, xr, src, dst):
    E = src.shape[0]
    PW = E // NW
    CO, KI = 200, 40
    NI = CO // KI
    NO = PW // CO
    assert NO % 2 == 0

    def body(xl_hbm, xr_hbm, src_hbm, dst_hbm,
             gl_hbm, gr_hbm,
             srcv0, dstv0, glv0, grv0, srcv1, dstv1, glv1, grv1,
             isem, gsem1, gsem2, dsem0, dsem1):
        c = lax.axis_index("c")
        sid = lax.axis_index("s")
        wid = sid * NC + c
        bufs = ((srcv0, dstv0, glv0, grv0, dsem0),
                (srcv1, dstv1, glv1, grv1, dsem1))

        def outer(o2, carry):
            for b, (sv, dv, gv, rv, dsem) in enumerate(bufs):
                o = o2 * 2 + b
                base = wid * PW + o * CO

                @pl.when(o2 > 0)
                def _():
                    pltpu.make_async_copy(
                        gv, gl_hbm.at[pl.ds(base, CO)], dsem).wait()
                    pltpu.make_async_copy(
                        rv, gr_hbm.at[pl.ds(base, CO)], dsem).wait()

                cpa = pltpu.async_copy(src_hbm.at[pl.ds(base, CO)], sv, isem)
                cpb = pltpu.async_copy(dst_hbm.at[pl.ds(base, CO)], dv, isem)
                cpa.wait()
                cpb.wait()
                cps = []
                for j in range(NI):
                    cps.append(pltpu.async_copy(
                        xl_hbm.at[sv.at[pl.ds(j * KI, KI)]],
                        gv.at[pl.ds(j * KI, KI)], gsem1))
                    cps.append(pltpu.async_copy(
                        xr_hbm.at[dv.at[pl.ds(j * KI, KI)]],
                        rv.at[pl.ds(j * KI, KI)], gsem2))
                for cp in cps:
                    cp.wait()
                pltpu.async_copy(gv, gl_hbm.at[pl.ds(base, CO)], dsem)
                pltpu.async_copy(rv, gr_hbm.at[pl.ds(base, CO)], dsem)
            return carry

        lax.fori_loop(0, NO // 2, outer, 0)
        for b, (sv, dv, gv, rv, dsem) in enumerate(bufs):
            pltpu.make_async_copy(gv, gl_hbm.at[pl.ds(0, CO)], dsem).wait()
            pltpu.make_async_copy(rv, gr_hbm.at[pl.ds(0, CO)], dsem).wait()

    f = pl.kernel(
        body,
        out_type=(jax.ShapeDtypeStruct((E, HD), F32),
                  jax.ShapeDtypeStruct((E, HD), F32)),
        mesh=_sc_mesh(),
        scratch_types=[pltpu.VMEM((CO,), jnp.int32),
                       pltpu.VMEM((CO,), jnp.int32),
                       pltpu.VMEM((CO, HD), F32),
                       pltpu.VMEM((CO, HD), F32),
                       pltpu.VMEM((CO,), jnp.int32),
                       pltpu.VMEM((CO,), jnp.int32),
                       pltpu.VMEM((CO, HD), F32),
                       pltpu.VMEM((CO, HD), F32),
                       pltpu.SemaphoreType.DMA,
                       pltpu.SemaphoreType.DMA,
                       pltpu.SemaphoreType.DMA,
                       pltpu.SemaphoreType.DMA,
                       pltpu.SemaphoreType.DMA])
    return f(xl, xr, src, dst)


def _sc_scatter_rows(val, dst2d, npad):
    KI = 80
    E = dst2d.shape[0] * KI
    CO = 320
    NI = CO // KI
    NCH = E // CO
    NK = _cdiv(NCH, NW)
    RPT = npad // NSUB
    ZR = 32

    def body(val_hbm, dst_hbm, out_hbm, valv, dstv, zv, tab, lsem, ssem):
        c = lax.axis_index("c")
        sid = lax.axis_index("s")
        wid = sid * NC + c
        _fill_const128(zv, ZR, 0.0)
        r0 = sid * RPT

        def zb(i, carry):
            pltpu.sync_copy(zv, tab.at[pl.ds(r0 + i * ZR, ZR)])
            return carry

        lax.fori_loop(0, RPT // ZR, zb, 0)
        plsc.subcore_barrier()

        def outer(k, carry):
            chunk = wid + k * NW

            @pl.when(chunk < NCH)
            def _():
                base = chunk * CO
                cpa = pltpu.async_copy(
                    dst_hbm.at[pl.ds(chunk * NI, NI)], dstv, lsem)
                cpb = pltpu.async_copy(val_hbm.at[pl.ds(base, CO)], valv, lsem)
                cpa.wait()
                cpb.wait()
                cps = []
                for j in range(NI):
                    cps.append(pltpu.async_copy(
                        valv.at[pl.ds(j * KI, KI)],
                        tab.at[dstv.at[j]], ssem, add=True))
                for cp in cps:
                    cp.wait()
            return carry

        lax.fori_loop(0, NK, outer, 0)
        plsc.subcore_barrier()
        pltpu.sync_copy(tab.at[pl.ds(r0, RPT)], out_hbm.at[c, pl.ds(r0, RPT)])

    f = pl.kernel(
        body,
        out_type=jax.ShapeDtypeStruct((NC, npad, HD), F32),
        mesh=_sc_mesh(),
        scratch_types=[pltpu.VMEM((CO, HD), F32),
                       pltpu.VMEM((NI, KI), jnp.int32),
                       pltpu.VMEM((ZR, HD), F32),
                       pltpu.VMEM_SHARED((npad, HD), F32),
                       pltpu.SemaphoreType.DMA,
                       pltpu.SemaphoreType.DMA])
    return f(val, dst2d)


def _sc_het(xsrc, srcp, dst2d, npad):
    KI = 80
    EP = dst2d.shape[0] * KI
    PW = EP // NW
    NI = PW // KI
    RPT = npad // NSUB
    ZR = 32

    def body(xs_hbm, src_hbm, dst_hbm, agg_hbm,
             srcv, dstv, rows, zv, tab, gsem, ssem):
        c = lax.axis_index("c")
        sid = lax.axis_index("s")
        wid = sid * NC + c
        _fill_const128(zv, ZR, 0.0)
        r0 = sid * RPT

        def zb(i, carry):
            pltpu.sync_copy(zv, tab.at[pl.ds(r0 + i * ZR, ZR)])
            return carry

        lax.fori_loop(0, RPT // ZR, zb, 0)
        plsc.subcore_barrier()

        base = wid * PW
        cpa = pltpu.async_copy(src_hbm.at[pl.ds(base, PW)], srcv, gsem)
        cpb = pltpu.async_copy(dst_hbm.at[pl.ds(wid * NI, NI)], dstv, gsem)
        cpa.wait()
        cpb.wait()
        cps = [pltpu.async_copy(xs_hbm.at[srcv.at[pl.ds(j * KI, KI)]],
                                rows.at[pl.ds(j * KI, KI)], gsem)
               for j in range(NI)]
        for cp in cps:
            cp.wait()
        cps = [pltpu.async_copy(rows.at[pl.ds(j * KI, KI)],
                                tab.at[dstv.at[j]], ssem, add=True)
               for j in range(NI)]
        for cp in cps:
            cp.wait()
        plsc.subcore_barrier()
        pltpu.sync_copy(tab.at[pl.ds(r0, RPT)], agg_hbm.at[c, pl.ds(r0, RPT)])

    f = pl.kernel(
        body,
        out_type=jax.ShapeDtypeStruct((NC, npad, HD), F32),
        mesh=_sc_mesh(),
        scratch_types=[pltpu.VMEM((PW,), jnp.int32),
                       pltpu.VMEM((NI, KI), jnp.int32),
                       pltpu.VMEM((PW, HD), F32),
                       pltpu.VMEM((ZR, HD), F32),
                       pltpu.VMEM_SHARED((npad, HD), F32),
                       pltpu.SemaphoreType.DMA,
                       pltpu.SemaphoreType.DMA])
    return f(xsrc, srcp, dst2d)


def _egret_full(x, src, dst, dst2d, ea16, p, We16):
    N = x.shape[0]
    xl, xr = _prep(x, p['bn1_w'], p['bn1_b'], p['Wl'], p['bl'], p['Wr'], p['br'])
    gl, gr = _sc_gatv2_gather(xl, xr, src, dst)
    val, sm = _edge_compute(gl, gr, ea16, We16, p['att'])
    out2 = _sc_scatter_rows(val, dst2d, NPAD_BIG)
    sm2 = _sc_scatter_rows(sm, dst2d, NPAD_BIG)
    return _finalize(x, xl, xr, sm2[0, :N], sm2[1, :N],
                     out2[0, :N], out2[1, :N],
                     We16, p['att'], p['bias'], p['ebn_w'], p['ebn_b'],
                     p['eW'], p['eb'])


def _het_block(x_src, x_dst, srcp, dst2d, npad, p):
    Nd = x_dst.shape[0]
    agg = _sc_het(x_src, srcp, dst2d, npad)
    epsb = jnp.broadcast_to(1.0 + p['eps'], (1, HD)).astype(F32)
    return _het_fin(x_dst, agg[0, :Nd], agg[1, :Nd], epsb,
                    p['W1'], p['b1'], p['W2'], p['b2'],
                    p['ebn_w'], p['ebn_b'], p['eW'], p['eb'])


def kernel(x_protein, x_drug, x_supernode, edge_attr_pp, edge_attr_dd, params,
           edge_index_pp, edge_index_dd, edge_index_ps, edge_index_ds,
           edge_index_sp, edge_index_sd):
    n_p = x_protein.shape[0]
    n_d = x_drug.shape[0]
    n_s = x_supernode.shape[0]
    pr = params['proj']
    xp = _mm_bias(x_protein, pr['Wp'], pr['bp'])
    xd = _mm_bias(x_drug, pr['Wd'], pr['bd'])
    xs = _mm_bias(x_supernode, pr['Ws'], pr['bs'])

    ea_pp16 = edge_attr_pp
    ea_dd16 = jnp.pad(edge_attr_dd, ((0, 0), (0, 16 - edge_attr_dd.shape[1])))

    def pad_het(ei, dummy):
        ec = ei.shape[1]
        srcp = jnp.pad(ei[0], (0, EPAD_HET - ec))
        dstp = jnp.pad(ei[1], (0, EPAD_HET - ec), constant_values=dummy)
        return srcp, dstp.reshape(EPAD_HET // 80, 80)

    dst2d_pp = edge_index_pp[1].reshape(-1, 80)
    dst2d_dd = edge_index_dd[1].reshape(-1, 80)
    ps_src, ps_dst = pad_het(edge_index_ps, n_s)
    ds_src, ds_dst = pad_het(edge_index_ds, n_s)
    sp_src, sp_dst = pad_het(edge_index_sp, n_p)
    sd_src, sd_dst = pad_het(edge_index_sd, n_d)

    for lp in params['layers']:
        We_dd16 = jnp.pad(lp['dd']['We'],
                          ((0, 16 - lp['dd']['We'].shape[0]), (0, 0)))
        o_pp = _egret_full(xp, edge_index_pp[0], edge_index_pp[1], dst2d_pp,
                           ea_pp16, lp['pp'], lp['pp']['We'])
        o_dd = _egret_full(xd, edge_index_dd[0], edge_index_dd[1], dst2d_dd,
                           ea_dd16, lp['dd'], We_dd16)
        o_ps = _het_block(xp, xs, ps_src, ps_dst, NPAD_SMALL, lp['ps'])
        o_ds = _het_block(xd, xs, ds_src, ds_dst, NPAD_SMALL, lp['ds'])
        o_sp = _het_block(xs, xp, sp_src, sp_dst, NPAD_BIG, lp['sp'])
        o_sd = _het_block(xs, xd, sd_src, sd_dst, NPAD_BIG, lp['sd'])
        xp = _mix(o_pp, o_sp, lp['bn_p'][0], lp['bn_p'][1])
        xd = _mix(o_dd, o_sd, lp['bn_d'][0], lp['bn_d'][1])
        xs = _mix(o_ps, o_ds, lp['bn_s'][0], lp['bn_s'][1])
    return xp, xd, xs
